# Initial kernel scaffold; baseline (speedup 1.0000x reference)
import jax
import jax.numpy as jnp
from jax.experimental import pallas as pl


def kernel(x, table):
    raise NotImplementedError("write your pallas kernel here")


# speedup vs baseline: 9.9206x; 9.9206x over previous
import jax
import jax.numpy as jnp
from jax import lax
from jax.experimental import pallas as pl
from jax.experimental.pallas import tpu as pltpu
from jax.experimental.pallas import tpu_sc as plsc

B, C, H, W = 16, 3, 224, 224
HIDDEN = 32
NTOK = 256
ROWPAD = 33
BC = B * C
HW = H * W
NW = 32
CHUNK = HW // NW
VECS = CHUNK // 16


def _body(x_hbm, lut_hbm, out_hbm, lut, xb0, xb1, ob0, ob1,
          xs0, xs1, os0, os1):
    cid = lax.axis_index("c")
    sid = lax.axis_index("s")
    wid = sid * 2 + cid
    p0 = wid * CHUNK
    xbufs = (xb0, xb1)
    obufs = (ob0, ob1)
    xsems = (xs0, xs1)
    osems = (os0, os1)

    pltpu.sync_copy(lut_hbm, lut)
    pltpu.async_copy(x_hbm.at[0, pl.ds(p0, CHUNK)], xb0, xs0)

    def compute(xbuf, obuf):
        def vbody(v, carry):
            base = xbuf[pl.ds(v * 16, 16)] * ROWPAD
            for d in range(HIDDEN):
                vals = plsc.load_gather(lut, [base + d])
                obuf[d, pl.ds(v * 16, 16)] = vals
            return carry
        lax.fori_loop(0, VECS, vbody, 0)

    def half(bh, carry):
        for par in range(2):
            bc = bh * 2 + par
            xbuf, obuf = xbufs[par], obufs[par]
            xsem, osem = xsems[par], osems[par]
            pltpu.make_async_copy(
                x_hbm.at[bc, pl.ds(p0, CHUNK)], xbuf, xsem).wait()

            @pl.when(bc < BC - 1)
            def _():
                pltpu.async_copy(
                    x_hbm.at[bc + 1, pl.ds(p0, CHUNK)],
                    xbufs[1 - par], xsems[1 - par])

            @pl.when(bh > 0)
            def _():
                pltpu.make_async_copy(
                    obuf, out_hbm.at[bc, :, pl.ds(p0, CHUNK)], osem).wait()

            compute(xbuf, obuf)
            pltpu.async_copy(
                obuf, out_hbm.at[bc, :, pl.ds(p0, CHUNK)], osem)
        return carry

    lax.fori_loop(0, BC // 2, half, 0)
    pltpu.make_async_copy(
        ob0, out_hbm.at[BC - 2, :, pl.ds(p0, CHUNK)], os0).wait()
    pltpu.make_async_copy(
        ob1, out_hbm.at[BC - 1, :, pl.ds(p0, CHUNK)], os1).wait()


@jax.jit
def _run(xf, tpad):
    f = pl.kernel(
        _body,
        out_type=jax.ShapeDtypeStruct((BC, HIDDEN, HW), jnp.float32),
        mesh=plsc.VectorSubcoreMesh(core_axis_name="c", subcore_axis_name="s"),
        scratch_types=[
            pltpu.VMEM((NTOK * ROWPAD,), jnp.float32),
            pltpu.VMEM((CHUNK,), jnp.int32),
            pltpu.VMEM((CHUNK,), jnp.int32),
            pltpu.VMEM((HIDDEN, CHUNK), jnp.float32),
            pltpu.VMEM((HIDDEN, CHUNK), jnp.float32),
            pltpu.SemaphoreType.DMA,
            pltpu.SemaphoreType.DMA,
            pltpu.SemaphoreType.DMA,
            pltpu.SemaphoreType.DMA,
        ],
        compiler_params=pltpu.CompilerParams(
            use_tc_tiling_on_sc=False, needs_layout_passes=False),
    )
    return f(xf, tpad)


def kernel(x, table):
    xf = x.reshape(BC, HW).astype(jnp.int32)
    tpad = jnp.pad(table, ((0, 0), (0, ROWPAD - HIDDEN))).reshape(-1)
    out = _run(xf, tpad)
    return out.reshape(B, C * HIDDEN, H, W)


# speedup vs baseline: 17.4957x; 1.7636x over previous
import jax
import jax.numpy as jnp
from jax import lax
from jax.experimental import pallas as pl
from jax.experimental.pallas import tpu as pltpu
from jax.experimental.pallas import tpu_sc as plsc

B, C, H, W = 16, 3, 224, 224
HIDDEN = 32
NTOK = 256
ROWPAD = 33
BC = B * C
HW = H * W
NW = 32
CHUNK = HW // NW
VECS = CHUNK // 16


def _body(x_hbm, lut_hbm, out_hbm, lut, xb0, xb1, ob0, ob1,
          xs0, xs1, os0, os1):
    cid = lax.axis_index("c")
    sid = lax.axis_index("s")
    wid = sid * 2 + cid
    p0 = wid * CHUNK
    xbufs = (xb0, xb1)
    obufs = (ob0, ob1)
    xsems = (xs0, xs1)
    osems = (os0, os1)

    pltpu.sync_copy(lut_hbm, lut)
    pltpu.async_copy(x_hbm.at[0, pl.ds(p0, CHUNK)], xb0, xs0)

    def compute(xbuf, obuf):
        @plsc.parallel_loop(0, VECS, 1, unroll=2)
        def vbody(v):
            base = xbuf[pl.ds(v * 16, 16)] * ROWPAD
            for d in range(HIDDEN):
                vals = plsc.load_gather(lut, [base + d])
                obuf[d, pl.ds(v * 16, 16)] = vals

    def half(bh, carry):
        for par in range(2):
            bc = bh * 2 + par
            xbuf, obuf = xbufs[par], obufs[par]
            xsem, osem = xsems[par], osems[par]
            pltpu.make_async_copy(
                x_hbm.at[bc, pl.ds(p0, CHUNK)], xbuf, xsem).wait()

            @pl.when(bc < BC - 1)
            def _():
                pltpu.async_copy(
                    x_hbm.at[bc + 1, pl.ds(p0, CHUNK)],
                    xbufs[1 - par], xsems[1 - par])

            @pl.when(bh > 0)
            def _():
                pltpu.make_async_copy(
                    obuf, out_hbm.at[bc, :, pl.ds(p0, CHUNK)], osem).wait()

            compute(xbuf, obuf)
            pltpu.async_copy(
                obuf, out_hbm.at[bc, :, pl.ds(p0, CHUNK)], osem)
        return carry

    lax.fori_loop(0, BC // 2, half, 0)
    pltpu.make_async_copy(
        ob0, out_hbm.at[BC - 2, :, pl.ds(p0, CHUNK)], os0).wait()
    pltpu.make_async_copy(
        ob1, out_hbm.at[BC - 1, :, pl.ds(p0, CHUNK)], os1).wait()


@jax.jit
def _run(xf, tpad):
    f = pl.kernel(
        _body,
        out_type=jax.ShapeDtypeStruct((BC, HIDDEN, HW), jnp.float32),
        mesh=plsc.VectorSubcoreMesh(core_axis_name="c", subcore_axis_name="s"),
        scratch_types=[
            pltpu.VMEM((NTOK * ROWPAD,), jnp.float32),
            pltpu.VMEM((CHUNK,), jnp.int32),
            pltpu.VMEM((CHUNK,), jnp.int32),
            pltpu.VMEM((HIDDEN, CHUNK), jnp.float32),
            pltpu.VMEM((HIDDEN, CHUNK), jnp.float32),
            pltpu.SemaphoreType.DMA,
            pltpu.SemaphoreType.DMA,
            pltpu.SemaphoreType.DMA,
            pltpu.SemaphoreType.DMA,
        ],
        compiler_params=pltpu.CompilerParams(
            use_tc_tiling_on_sc=False, needs_layout_passes=False),
    )
    return f(xf, tpad)


def kernel(x, table):
    xf = x.reshape(BC, HW).astype(jnp.int32)
    tpad = jnp.pad(table, ((0, 0), (0, ROWPAD - HIDDEN))).reshape(-1)
    out = _run(xf, tpad)
    return out.reshape(B, C * HIDDEN, H, W)


# speedup vs baseline: 53.1069x; 3.0354x over previous
import jax
import jax.numpy as jnp
from jax import lax
from jax.experimental import pallas as pl
from jax.experimental.pallas import tpu as pltpu
from jax.experimental.pallas import tpu_sc as plsc

B, C, H, W = 16, 3, 224, 224
HIDDEN = 32
NTOK = 256
ROWPAD = 33
BC = B * C
NW = 32
HSTRIP = 8
NSTRIP = H // HSTRIP
UNITS = BC * NSTRIP
UPT = UNITS // NW
WVECS = W // 16
DCH = HIDDEN // 2


def _div28(u):
    return (u * 2341) >> 16


def _body(x_hbm, lut_hbm, out_hbm, lut, xb0, xb1, ob0, ob1,
          xs0, xs1, os0, os1):
    cid = lax.axis_index("c")
    sid = lax.axis_index("s")
    tid = sid * 2 + cid
    u0 = tid * UPT
    xbufs = (xb0, xb1)
    obufs = (ob0, ob1)
    xsems = (xs0, xs1)
    osems = (os0, os1)

    def xslice(u):
        bc = _div28(u)
        ht = u - bc * NSTRIP
        return x_hbm.at[bc, pl.ds(ht * HSTRIP, HSTRIP), :]

    def oslice(u, half):
        bc = _div28(u)
        ht = u - bc * NSTRIP
        return out_hbm.at[pl.ds(bc * HIDDEN + half * DCH, DCH),
                          pl.ds(ht * HSTRIP, HSTRIP), :]

    pltpu.sync_copy(lut_hbm, lut)
    pltpu.async_copy(xslice(u0), xb0, xs0)

    def compute_half(xbuf, obuf, half):
        def hbody(h, carry):
            @plsc.parallel_loop(0, WVECS, 1, unroll=2)
            def wbody(wc):
                base = xbuf[h, pl.ds(wc * 16, 16)] * ROWPAD + half * DCH
                for d in range(DCH):
                    vals = plsc.load_gather(lut, [base + d])
                    obuf[d, h, pl.ds(wc * 16, 16)] = vals
            return carry
        lax.fori_loop(0, HSTRIP, hbody, 0)

    def pair(k, carry):
        for par in range(2):
            r = k * 2 + par
            u = u0 + r
            xbuf = xbufs[par]
            pltpu.make_async_copy(xslice(u), xbuf, xsems[par]).wait()

            @pl.when(r < UPT - 1)
            def _():
                pltpu.async_copy(xslice(u + 1), xbufs[1 - par],
                                 xsems[1 - par])

            for half in range(2):
                @pl.when(r > 0)
                def _():
                    pltpu.make_async_copy(
                        obufs[half], oslice(u, half), osems[half]).wait()

                compute_half(xbuf, obufs[half], half)
                pltpu.async_copy(obufs[half], oslice(u, half), osems[half])
        return carry

    lax.fori_loop(0, UPT // 2, pair, 0)
    pltpu.make_async_copy(ob0, oslice(u0 + UPT - 1, 0), os0).wait()
    pltpu.make_async_copy(ob1, oslice(u0 + UPT - 1, 1), os1).wait()


@jax.jit
def _run(xf, tpad):
    f = pl.kernel(
        _body,
        out_type=jax.ShapeDtypeStruct((BC * HIDDEN, H, W), jnp.float32),
        mesh=plsc.VectorSubcoreMesh(core_axis_name="c", subcore_axis_name="s"),
        scratch_types=[
            pltpu.VMEM((NTOK * ROWPAD,), jnp.float32),
            pltpu.VMEM((HSTRIP, W), jnp.int32),
            pltpu.VMEM((HSTRIP, W), jnp.int32),
            pltpu.VMEM((DCH, HSTRIP, W), jnp.float32),
            pltpu.VMEM((DCH, HSTRIP, W), jnp.float32),
            pltpu.SemaphoreType.DMA,
            pltpu.SemaphoreType.DMA,
            pltpu.SemaphoreType.DMA,
            pltpu.SemaphoreType.DMA,
        ],
        compiler_params=pltpu.CompilerParams(needs_layout_passes=False),
    )
    return f(xf, tpad)


def kernel(x, table):
    xf = x.reshape(BC, H, W).astype(jnp.int32)
    tpad = jnp.pad(table, ((0, 0), (0, ROWPAD - HIDDEN))).reshape(-1)
    out = _run(xf, tpad)
    return out.reshape(B, C * HIDDEN, H, W)
